# baseline (speedup 1.0000x reference)
import functools

import numpy as np

import jax
import jax.numpy as jnp
from jax.experimental import pallas as pl

_CW, _CH = 8, 8


@functools.lru_cache(maxsize=None)
def _pool_matrix(w: int, h: int) -> np.ndarray:
    n = np.arange(w * h)
    r, col = n // h, n % h
    m = (r // (w // _CW)) * _CH + col // (h // _CH)
    P = np.zeros((w * h, _CW * _CH), dtype=np.float32)
    P[n, m] = 1.0
    return P


def _threefry2x32(k0, k1, x0, x1):
    rotations = ((13, 15, 26, 6), (17, 29, 16, 24))
    rotl = lambda v, r: ((v << np.uint32(r)) | (v >> np.uint32(32 - r)))
    ks = (k0, k1, k0 ^ k1 ^ np.uint32(0x1BD11BDA))
    x0, x1 = x0 + ks[0], x1 + ks[1]
    for i in range(5):
        for r in rotations[i % 2]:
            x0 = x0 + x1
            x1 = rotl(x1, r) ^ x0
        x0 = x0 + ks[(i + 1) % 3]
        x1 = x1 + ks[(i + 2) % 3] + np.uint32(i + 1)
    return x0, x1


@functools.lru_cache(maxsize=None)
def _drop_uniform(b: int, m: int) -> np.ndarray:
    size = b * m
    lo = np.arange(size, dtype=np.uint32)
    with np.errstate(over="ignore"):
        h0, h1 = _threefry2x32(np.uint32(0), np.uint32(42),
                               np.zeros(size, np.uint32), lo)
    bits = h0 ^ h1
    f = ((bits >> np.uint32(9)) | np.uint32(0x3F800000)).view(np.float32) - 1.0
    return np.maximum(f, 0.0).reshape(b, m)


def _one_batch(xb, pool, lm_row, u_col):
    cen = jnp.dot(xb, pool, preferred_element_type=jnp.float32,
                  precision=jax.lax.Precision.HIGHEST) * (1.0 / 16.0)
    norm = jnp.sqrt(jnp.sum(cen * cen, axis=0, keepdims=True))
    cen_n = cen / jnp.maximum(norm, 1e-12)
    xnorm = jnp.sqrt(jnp.sum(xb * xb, axis=0, keepdims=True))
    xb_n = xb / jnp.maximum(xnorm, 1e-12)
    sim = jax.lax.dot_general(
        cen_n, xb_n, dimension_numbers=(((0,), (0,)), ((), ())),
        preferred_element_type=jnp.float32,
        precision=jax.lax.Precision.HIGHEST)
    sim = jax.nn.sigmoid(sim)
    M, N = sim.shape
    mx = jnp.max(sim, axis=0, keepdims=True)
    iota_m = jax.lax.broadcasted_iota(jnp.int32, (M, N), 0)
    idx = jnp.min(jnp.where(sim == mx, iota_m, M), axis=0, keepdims=True)
    oh = (iota_m == idx).astype(jnp.float32)
    two = jnp.concatenate([lm_row, jnp.ones_like(lm_row)], axis=0)
    sc = jax.lax.dot_general(
        oh, two, dimension_numbers=(((1,), (1,)), ((), ())),
        preferred_element_type=jnp.float32,
        precision=jax.lax.Precision.HIGHEST)
    logit = sc[:, 0:1] / (sc[:, 1:2] + 1e-6)
    keep_p = jnp.clip(1.0 - jnp.maximum(logit, 0.0), 0.0, 1.0)
    drop = (u_col < keep_p).astype(jnp.float32)
    return jnp.sum(oh * drop, axis=0, keepdims=True)


def _body(x_ref, p_ref, lm_ref, u_ref, o_ref):
    bb = x_ref.shape[0]
    pool = p_ref[...]
    for j in range(bb):
        o_ref[j] = _one_batch(x_ref[j], pool, lm_ref[j], u_ref[j])


def kernel(x, logit_mask):
    b, c, w, h = x.shape
    N, M = w * h, _CW * _CH
    BB = 2
    x2 = x.reshape(b, c, N)
    lm3 = logit_mask.reshape(b, 1, N)
    u3 = jnp.asarray(_drop_uniform(b, M)).reshape(b, M, 1)
    P = jnp.asarray(_pool_matrix(w, h))
    out = pl.pallas_call(
        _body,
        grid=(b // BB,),
        in_specs=[
            pl.BlockSpec((BB, c, N), lambda i: (i, 0, 0)),
            pl.BlockSpec((N, M), lambda i: (0, 0)),
            pl.BlockSpec((BB, 1, N), lambda i: (i, 0, 0)),
            pl.BlockSpec((BB, M, 1), lambda i: (i, 0, 0)),
        ],
        out_specs=pl.BlockSpec((BB, 1, N), lambda i: (i, 0, 0)),
        out_shape=jax.ShapeDtypeStruct((b, 1, N), jnp.float32),
    )(x2, P, lm3, u3)
    return out.reshape(b, w, h)


# speedup vs baseline: 2.4418x; 2.4418x over previous
import functools

import numpy as np

import jax
import jax.numpy as jnp
from jax.experimental import pallas as pl

_CW, _CH = 8, 8


@functools.lru_cache(maxsize=None)
def _pool_matrix(w: int, h: int) -> np.ndarray:
    n = np.arange(w * h)
    r, col = n // h, n % h
    m = (r // (w // _CW)) * _CH + col // (h // _CH)
    P = np.zeros((w * h, _CW * _CH), dtype=np.float32)
    P[n, m] = 1.0
    return P


def _threefry2x32(k0, k1, x0, x1):
    rotations = ((13, 15, 26, 6), (17, 29, 16, 24))
    rotl = lambda v, r: ((v << np.uint32(r)) | (v >> np.uint32(32 - r)))
    ks = (k0, k1, k0 ^ k1 ^ np.uint32(0x1BD11BDA))
    x0, x1 = x0 + ks[0], x1 + ks[1]
    for i in range(5):
        for r in rotations[i % 2]:
            x0 = x0 + x1
            x1 = rotl(x1, r) ^ x0
        x0 = x0 + ks[(i + 1) % 3]
        x1 = x1 + ks[(i + 2) % 3] + np.uint32(i + 1)
    return x0, x1


@functools.lru_cache(maxsize=None)
def _drop_uniform(b: int, m: int) -> np.ndarray:
    size = b * m
    lo = np.arange(size, dtype=np.uint32)
    with np.errstate(over="ignore"):
        h0, h1 = _threefry2x32(np.uint32(0), np.uint32(42),
                               np.zeros(size, np.uint32), lo)
    bits = h0 ^ h1
    f = ((bits >> np.uint32(9)) | np.uint32(0x3F800000)).view(np.float32) - 1.0
    return np.maximum(f, 0.0).reshape(b, m)


def _one_batch(xb, pool, lm_row, u_col):
    cen = jnp.dot(xb, pool, preferred_element_type=jnp.float32) * (1.0 / 16.0)
    norm = jnp.sqrt(jnp.sum(cen * cen, axis=0, keepdims=True))
    cen_n = cen / jnp.maximum(norm, 1e-12)
    xnorm = jnp.sqrt(jnp.sum(xb * xb, axis=0, keepdims=True))
    xb_n = xb / jnp.maximum(xnorm, 1e-12)
    sim = jax.lax.dot_general(
        cen_n, xb_n, dimension_numbers=(((0,), (0,)), ((), ())),
        preferred_element_type=jnp.float32)
    sim = jax.nn.sigmoid(sim)
    M, N = sim.shape
    mx = jnp.max(sim, axis=0, keepdims=True)
    iota_m = jax.lax.broadcasted_iota(jnp.int32, (M, N), 0)
    idx = jnp.min(jnp.where(sim == mx, iota_m, M), axis=0, keepdims=True)
    oh = (iota_m == idx).astype(jnp.float32)
    two = jnp.concatenate([lm_row, jnp.ones_like(lm_row)], axis=0)
    sc = jax.lax.dot_general(
        oh, two, dimension_numbers=(((1,), (1,)), ((), ())),
        preferred_element_type=jnp.float32,
        precision=jax.lax.Precision.HIGHEST)
    logit = sc[:, 0:1] / (sc[:, 1:2] + 1e-6)
    keep_p = jnp.clip(1.0 - jnp.maximum(logit, 0.0), 0.0, 1.0)
    drop = (u_col < keep_p).astype(jnp.float32)
    return jnp.sum(oh * drop, axis=0, keepdims=True)


def _body(x_ref, p_ref, lm_ref, u_ref, o_ref):
    bb = x_ref.shape[0]
    pool = p_ref[...]
    for j in range(bb):
        o_ref[j] = _one_batch(x_ref[j], pool, lm_ref[j], u_ref[j])


def kernel(x, logit_mask):
    b, c, w, h = x.shape
    N, M = w * h, _CW * _CH
    BB = 2
    x2 = x.reshape(b, c, N)
    lm3 = logit_mask.reshape(b, 1, N)
    u3 = jnp.asarray(_drop_uniform(b, M)).reshape(b, M, 1)
    P = jnp.asarray(_pool_matrix(w, h))
    out = pl.pallas_call(
        _body,
        grid=(b // BB,),
        in_specs=[
            pl.BlockSpec((BB, c, N), lambda i: (i, 0, 0)),
            pl.BlockSpec((N, M), lambda i: (0, 0)),
            pl.BlockSpec((BB, 1, N), lambda i: (i, 0, 0)),
            pl.BlockSpec((BB, M, 1), lambda i: (i, 0, 0)),
        ],
        out_specs=pl.BlockSpec((BB, 1, N), lambda i: (i, 0, 0)),
        out_shape=jax.ShapeDtypeStruct((b, 1, N), jnp.float32),
    )(x2, P, lm3, u3)
    return out.reshape(b, w, h)
